# baseline (speedup 1.0000x reference)
import jax
import jax.numpy as jnp
from jax import lax
from jax.experimental import pallas as pl
from jax.experimental.pallas import tpu as pltpu
from jax.experimental.pallas import tpu_sc as plsc

_B = 1024
_L = 200
_WDIM = 64
_PDIM = 16
_P2 = 2 * _PDIM
_XDIM = _WDIM + _P2
_N = _B * _L
_NC = 2
_NS = 16
_NW = _NC * _NS
_PER_W = _N // _NW
_C = 128
_G = 256
_NG = _PER_W // _G
_NB = 3
_K = 2
_HT_PER_W = _B // _NW

_SUBS = [(o, min(_C, _G - o)) for o in range(0, _G, _C)]
_VISITS = _NG + _NB
_OUTER = -(-_VISITS // _NB)


def _sc_body(word_hbm, pos1_hbm, pos2_hbm, m0_hbm, head_hbm, tail_hbm,
             wtab_hbm, p1tab_hbm, p2tab_hbm,
             x_hbm, head_out_hbm, tail_out_hbm,
             widx, pidx1, pidx2, m0t, wbuf, pbuf, p1tab_v, p2tab_v,
             hidx_v, hbuf, gsem, wsem, hsem):
    wid = lax.axis_index("s") * _NC + lax.axis_index("c")
    base = wid * _PER_W

    hbase = wid * _HT_PER_W
    pltpu.sync_copy(head_hbm.at[pl.ds(hbase, _HT_PER_W)], hidx_v)
    c1 = pltpu.async_copy(p1tab_hbm, p1tab_v.at[:, pl.ds(0, _PDIM)], hsem)
    c2 = pltpu.async_copy(p2tab_hbm, p2tab_v.at[:, pl.ds(0, _PDIM)], hsem)
    c3 = pltpu.async_copy(word_hbm.at[pl.ds(base, _PER_W)], widx, hsem)
    c4 = pltpu.async_copy(pos1_hbm.at[pl.ds(base, _PER_W)], pidx1, hsem)
    c5 = pltpu.async_copy(pos2_hbm.at[pl.ds(base, _PER_W)], pidx2, hsem)
    c6 = pltpu.async_copy(m0_hbm.at[pl.ds(base, _PER_W)], m0t, hsem)
    ch = pltpu.async_copy(wtab_hbm.at[hidx_v], hbuf, hsem)
    for c in (c1, c2, c3, c4, c5, c6, ch):
        c.wait()
    pltpu.sync_copy(hbuf, head_out_hbm.at[pl.ds(hbase, _HT_PER_W)])
    pltpu.sync_copy(tail_hbm.at[pl.ds(hbase, _HT_PER_W)], hidx_v)
    pltpu.async_copy(wtab_hbm.at[hidx_v], hbuf, hsem).wait()
    pltpu.sync_copy(hbuf, tail_out_hbm.at[pl.ds(hbase, _HT_PER_W)])

    def gather_copies(g, b):
        cs = []
        for off, n in _SUBS:
            tok = g * _G + off
            cs.append(pltpu.make_async_copy(
                wtab_hbm.at[widx.at[pl.ds(tok, n)]],
                wbuf.at[b, pl.ds(off, n)], gsem.at[b]))
        return cs

    def write_copies(g, b):
        off = base + g * _G
        return [
            pltpu.make_async_copy(
                wbuf.at[b], x_hbm.at[pl.ds(off, _G), pl.ds(0, _WDIM)],
                wsem.at[b]),
            pltpu.make_async_copy(
                pbuf.at[b, :, pl.ds(0, _P2)],
                x_hbm.at[pl.ds(off, _G), pl.ds(_WDIM, _P2)], wsem.at[b]),
        ]

    lanes = lax.iota(jnp.int32, 16)

    def pos_group(g, b):
        tok0 = g * _G

        def tloop(t, carry):
            s = pl.ds(tok0 + t * 16, 16)
            m = m0t[s]
            r1 = pidx1[s] * m
            r2 = pidx2[s] * m
            trow = lanes + t * 16
            for j in range(_PDIM):
                cj = jnp.full((16,), j, jnp.int32)
                v1 = plsc.load_gather(p1tab_v, [r1, cj])
                plsc.store_scatter(pbuf.at[b], [trow, cj], v1)
                v2 = plsc.load_gather(p2tab_v, [r2, cj])
                plsc.store_scatter(pbuf.at[b], [trow, cj + _PDIM], v2)
            return carry

        lax.fori_loop(0, _G // 16, tloop, 0)

    def outer(o, carry):
        for b in range(_NB):
            i = o * _NB + b

            @pl.when(jnp.logical_and(i >= _NB, i < _NG + _NB))
            def _():
                for c in write_copies(i - _NB, b):
                    c.wait()

            @pl.when(i < _NG)
            def _():
                for c in gather_copies(i, b):
                    c.start()
                pos_group(i, b)

            bj = (b - _K) % _NB

            @pl.when(jnp.logical_and(i >= _K, i < _NG + _K))
            def _():
                for c in gather_copies(i - _K, bj):
                    c.wait()
                for c in write_copies(i - _K, bj):
                    c.start()

        return carry

    lax.fori_loop(0, _OUTER, outer, 0)


def kernel(word, pos1, pos2, mask, mask0, head, tail,
           word_table, pos1_table, pos2_table):
    del mask
    word_f = word.reshape(_N).astype(jnp.int32)
    pos1_f = pos1.reshape(_N).astype(jnp.int32)
    pos2_f = pos2.reshape(_N).astype(jnp.int32)
    m0_f = mask0.reshape(_N).astype(jnp.int32)
    head_i = head.astype(jnp.int32)
    tail_i = tail.astype(jnp.int32)

    mesh = plsc.VectorSubcoreMesh(core_axis_name="c", subcore_axis_name="s",
                                  num_cores=_NC, num_subcores=_NS)
    x_flat, head_e, tail_e = pl.kernel(
        _sc_body,
        out_type=(
            jax.ShapeDtypeStruct((_N, _XDIM), jnp.float32),
            jax.ShapeDtypeStruct((_B, _WDIM), jnp.float32),
            jax.ShapeDtypeStruct((_B, _WDIM), jnp.float32),
        ),
        mesh=mesh,
        compiler_params=pltpu.CompilerParams(use_tc_tiling_on_sc=False,
                                             needs_layout_passes=False),
        scratch_types=[
            pltpu.VMEM((_PER_W,), jnp.int32),
            pltpu.VMEM((_PER_W,), jnp.int32),
            pltpu.VMEM((_PER_W,), jnp.int32),
            pltpu.VMEM((_PER_W,), jnp.int32),
            pltpu.VMEM((_NB, _G, _WDIM), jnp.float32),
            pltpu.VMEM((_NB, _G, _P2 + 1), jnp.float32),
            pltpu.VMEM((80, _PDIM + 1), jnp.float32),
            pltpu.VMEM((80, _PDIM + 1), jnp.float32),
            pltpu.VMEM((_HT_PER_W,), jnp.int32),
            pltpu.VMEM((_HT_PER_W, _WDIM), jnp.float32),
            pltpu.SemaphoreType.DMA((_NB,)),
            pltpu.SemaphoreType.DMA((_NB,)),
            pltpu.SemaphoreType.DMA,
        ],
    )(word_f, pos1_f, pos2_f, m0_f, head_i, tail_i,
      word_table, pos1_table, pos2_table)
    return x_flat.reshape(_B, _L, _XDIM), head_e, tail_e


# speedup vs baseline: 1.0070x; 1.0070x over previous
import jax
import jax.numpy as jnp
from jax import lax
from jax.experimental import pallas as pl
from jax.experimental.pallas import tpu as pltpu
from jax.experimental.pallas import tpu_sc as plsc

_B = 1024
_L = 200
_WDIM = 64
_PDIM = 16
_P2 = 2 * _PDIM
_XDIM = _WDIM + _P2
_N = _B * _L
_NC = 2
_NS = 16
_NW = _NC * _NS
_PER_W = _N // _NW
_C = 128
_G = 128
_NG = _PER_W // _G
_NB = 6
_K = 2
_HT_PER_W = _B // _NW

_SUBS = [(o, min(_C, _G - o)) for o in range(0, _G, _C)]
_VISITS = _NG + _NB
_OUTER = -(-_VISITS // _NB)


def _sc_body(word_hbm, pos1_hbm, pos2_hbm, m0_hbm, head_hbm, tail_hbm,
             wtab_hbm, p1tab_hbm, p2tab_hbm,
             x_hbm, head_out_hbm, tail_out_hbm,
             widx, pidx1, pidx2, m0t, wbuf, pbuf, p1tab_v, p2tab_v,
             hidx_v, hbuf, gsem, wsem, hsem):
    wid = lax.axis_index("s") * _NC + lax.axis_index("c")
    base = wid * _PER_W

    hbase = wid * _HT_PER_W
    pltpu.sync_copy(head_hbm.at[pl.ds(hbase, _HT_PER_W)], hidx_v)
    c1 = pltpu.async_copy(p1tab_hbm, p1tab_v.at[:, pl.ds(0, _PDIM)], hsem)
    c2 = pltpu.async_copy(p2tab_hbm, p2tab_v.at[:, pl.ds(0, _PDIM)], hsem)
    c3 = pltpu.async_copy(word_hbm.at[pl.ds(base, _PER_W)], widx, hsem)
    c4 = pltpu.async_copy(pos1_hbm.at[pl.ds(base, _PER_W)], pidx1, hsem)
    c5 = pltpu.async_copy(pos2_hbm.at[pl.ds(base, _PER_W)], pidx2, hsem)
    c6 = pltpu.async_copy(m0_hbm.at[pl.ds(base, _PER_W)], m0t, hsem)
    ch = pltpu.async_copy(wtab_hbm.at[hidx_v], hbuf, hsem)
    for c in (c1, c2, c3, c4, c5, c6, ch):
        c.wait()
    pltpu.sync_copy(hbuf, head_out_hbm.at[pl.ds(hbase, _HT_PER_W)])
    pltpu.sync_copy(tail_hbm.at[pl.ds(hbase, _HT_PER_W)], hidx_v)
    pltpu.async_copy(wtab_hbm.at[hidx_v], hbuf, hsem).wait()
    pltpu.sync_copy(hbuf, tail_out_hbm.at[pl.ds(hbase, _HT_PER_W)])

    def gather_copies(g, b):
        cs = []
        for off, n in _SUBS:
            tok = g * _G + off
            cs.append(pltpu.make_async_copy(
                wtab_hbm.at[widx.at[pl.ds(tok, n)]],
                wbuf.at[b, pl.ds(off, n)], gsem.at[b]))
        return cs

    def write_copies(g, b):
        off = base + g * _G
        return [
            pltpu.make_async_copy(
                wbuf.at[b], x_hbm.at[pl.ds(off, _G), pl.ds(0, _WDIM)],
                wsem.at[b]),
            pltpu.make_async_copy(
                pbuf.at[b, :, pl.ds(0, _P2)],
                x_hbm.at[pl.ds(off, _G), pl.ds(_WDIM, _P2)], wsem.at[b]),
        ]

    lanes = lax.iota(jnp.int32, 16)

    def pos_group(g, b):
        tok0 = g * _G

        def tloop(t, carry):
            s = pl.ds(tok0 + t * 16, 16)
            m = m0t[s]
            r1 = pidx1[s] * m
            r2 = pidx2[s] * m
            trow = lanes + t * 16
            for j in range(_PDIM):
                cj = jnp.full((16,), j, jnp.int32)
                v1 = plsc.load_gather(p1tab_v, [r1, cj])
                plsc.store_scatter(pbuf.at[b], [trow, cj], v1)
                v2 = plsc.load_gather(p2tab_v, [r2, cj])
                plsc.store_scatter(pbuf.at[b], [trow, cj + _PDIM], v2)
            return carry

        lax.fori_loop(0, _G // 16, tloop, 0)

    def outer(o, carry):
        for b in range(_NB):
            i = o * _NB + b

            @pl.when(jnp.logical_and(i >= _NB, i < _NG + _NB))
            def _():
                for c in write_copies(i - _NB, b):
                    c.wait()

            @pl.when(i < _NG)
            def _():
                for c in gather_copies(i, b):
                    c.start()
                pos_group(i, b)

            bj = (b - _K) % _NB

            @pl.when(jnp.logical_and(i >= _K, i < _NG + _K))
            def _():
                for c in gather_copies(i - _K, bj):
                    c.wait()
                for c in write_copies(i - _K, bj):
                    c.start()

        return carry

    lax.fori_loop(0, _OUTER, outer, 0)


def kernel(word, pos1, pos2, mask, mask0, head, tail,
           word_table, pos1_table, pos2_table):
    del mask
    word_f = word.reshape(_N).astype(jnp.int32)
    pos1_f = pos1.reshape(_N).astype(jnp.int32)
    pos2_f = pos2.reshape(_N).astype(jnp.int32)
    m0_f = mask0.reshape(_N).astype(jnp.int32)
    head_i = head.astype(jnp.int32)
    tail_i = tail.astype(jnp.int32)

    mesh = plsc.VectorSubcoreMesh(core_axis_name="c", subcore_axis_name="s",
                                  num_cores=_NC, num_subcores=_NS)
    x_flat, head_e, tail_e = pl.kernel(
        _sc_body,
        out_type=(
            jax.ShapeDtypeStruct((_N, _XDIM), jnp.float32),
            jax.ShapeDtypeStruct((_B, _WDIM), jnp.float32),
            jax.ShapeDtypeStruct((_B, _WDIM), jnp.float32),
        ),
        mesh=mesh,
        compiler_params=pltpu.CompilerParams(use_tc_tiling_on_sc=False,
                                             needs_layout_passes=False),
        scratch_types=[
            pltpu.VMEM((_PER_W,), jnp.int32),
            pltpu.VMEM((_PER_W,), jnp.int32),
            pltpu.VMEM((_PER_W,), jnp.int32),
            pltpu.VMEM((_PER_W,), jnp.int32),
            pltpu.VMEM((_NB, _G, _WDIM), jnp.float32),
            pltpu.VMEM((_NB, _G, _P2 + 1), jnp.float32),
            pltpu.VMEM((80, _PDIM + 1), jnp.float32),
            pltpu.VMEM((80, _PDIM + 1), jnp.float32),
            pltpu.VMEM((_HT_PER_W,), jnp.int32),
            pltpu.VMEM((_HT_PER_W, _WDIM), jnp.float32),
            pltpu.SemaphoreType.DMA((_NB,)),
            pltpu.SemaphoreType.DMA((_NB,)),
            pltpu.SemaphoreType.DMA,
        ],
    )(word_f, pos1_f, pos2_f, m0_f, head_i, tail_i,
      word_table, pos1_table, pos2_table)
    return x_flat.reshape(_B, _L, _XDIM), head_e, tail_e
